# baseline (speedup 1.0000x reference)
import jax
import jax.numpy as jnp
from jax import lax
from jax.experimental import pallas as pl
from jax.experimental.pallas import tpu as pltpu
from jax.experimental.pallas import tpu_sc as plsc

NC = 2
NS = 16
NW = NC * NS
L = 16


def _sc_mesh():
    return plsc.VectorSubcoreMesh(core_axis_name="c", subcore_axis_name="s")


def _sc_degrees(eflat, hni, hei, E, n_pad, m_pad):
    EW = E // NW
    K = 80
    NB = 5
    assert EW % K == 0 and (EW // K) % NB == 0
    CN = n_pad // NS
    CM = m_pad // NS
    assert CN % 8 == 0 and CM % 8 == 0
    f32 = jnp.float32

    def body(eflat_hbm, hni_hbm, hei_hbm, deg_out, dv_out, de_out, *scr):
        idx_v = scr[0:NB]
        isem = scr[NB:2 * NB]
        ones_v, zero_v, s_n, s_v, s_e = scr[2 * NB:]
        cid = lax.axis_index("c")
        sid = lax.axis_index("s")
        wid = sid * NC + cid
        zero16 = jnp.zeros((L,), f32)
        one16 = jnp.ones((L,), f32)

        def fill(ref, n, vec):
            def f(i, _):
                ref[pl.ds(i * L, L)] = vec
                return 0
            lax.fori_loop(0, n // L, f, 0)

        fill(ones_v, K, one16)
        fill(zero_v, CN, zero16)

        pltpu.sync_copy(zero_v.at[pl.ds(0, CN)], s_n.at[pl.ds(sid * CN, CN)])
        pltpu.sync_copy(zero_v.at[pl.ds(0, CN)], s_v.at[pl.ds(sid * CN, CN)])
        pltpu.sync_copy(zero_v.at[pl.ds(0, CM)], s_e.at[pl.ds(sid * CM, CM)])
        plsc.subcore_barrier()

        base = wid * EW
        G = EW // K

        def count(idx_hbm, off0, shared):
            def fetch(b, g):
                pltpu.async_copy(idx_hbm.at[pl.ds(off0 + base + g * K, K)],
                                 idx_v[b], isem[b])

            for b in range(NB):
                fetch(b, b)

            def outer(o, _):
                for b in range(NB):
                    g = o * NB + b
                    pltpu.make_async_copy(idx_hbm.at[pl.ds(off0 + base, K)],
                                          idx_v[b], isem[b]).wait()
                    pltpu.sync_copy(ones_v, shared.at[idx_v[b]], add=True)

                    @pl.when(g + NB < G)
                    def _():
                        fetch(b, g + NB)
                return 0

            lax.fori_loop(0, G // NB, outer, 0)

        count(eflat_hbm, E, s_n)
        count(hni_hbm, 0, s_v)
        count(hei_hbm, 0, s_e)
        plsc.subcore_barrier()

        def copy_out(shared, out_ref, cw, pitch):
            pltpu.sync_copy(shared.at[pl.ds(sid * cw, cw)],
                            zero_v.at[pl.ds(0, cw)])
            pltpu.sync_copy(zero_v.at[pl.ds(0, cw)],
                            out_ref.at[pl.ds(cid * pitch + sid * cw, cw)])

        copy_out(s_n, deg_out, CN, n_pad)
        copy_out(s_v, dv_out, CN, n_pad)
        copy_out(s_e, de_out, CM, m_pad)

    fn = pl.kernel(
        body,
        out_type=(jax.ShapeDtypeStruct((NC * n_pad,), f32),
                  jax.ShapeDtypeStruct((NC * n_pad,), f32),
                  jax.ShapeDtypeStruct((NC * m_pad,), f32)),
        mesh=_sc_mesh(),
        scratch_types=(
            [pltpu.VMEM((K,), jnp.int32)] * NB
            + [pltpu.SemaphoreType.DMA] * NB
            + [pltpu.VMEM((K,), f32),
               pltpu.VMEM((CN,), f32),
               pltpu.VMEM_SHARED((n_pad,), f32),
               pltpu.VMEM_SHARED((n_pad,), f32),
               pltpu.VMEM_SHARED((m_pad,), f32)]
        ),
    )
    return fn(eflat, hni, hei)


def _sc_pass(table, src, dst, ndst_pad):
    nsrc, D = table.shape
    E = src.shape[0]
    EW = E // NW
    K = 80
    NB = 4
    G = EW // K
    GOUT = (G + NB - 1) // NB
    assert EW % K == 0
    RPT = ndst_pad // NS
    ZC = min(RPT, 40)
    nfull, tail = RPT // ZC, RPT % ZC
    f32 = jnp.float32

    NI = 2 * NB

    def body(tbl_hbm, src_hbm, dst_hbm, out_hbm, *scr):
        idxs_v = scr[0:NI]
        idxd_v = scr[NI:2 * NI]
        rows_v = scr[2 * NI:2 * NI + NB]
        zbuf_v = scr[2 * NI + NB]
        acc_s = scr[2 * NI + NB + 1]
        p = 2 * NI + NB + 2
        sems = scr[p:p + NB]
        isem_s = scr[p + NB:p + NB + NI]
        isem_d = scr[p + NB + NI:p + NB + 2 * NI]
        cid = lax.axis_index("c")
        sid = lax.axis_index("s")
        wid = sid * NC + cid
        zero16 = jnp.zeros((L,), f32)

        def zf(i, _):
            r = i // (D // L)
            c = i % (D // L)
            zbuf_v[r, pl.ds(c * L, L)] = zero16
            return 0

        lax.fori_loop(0, ZC * (D // L), zf, 0)

        base = sid * RPT
        for q in range(nfull):
            pltpu.sync_copy(zbuf_v, acc_s.at[pl.ds(base + q * ZC, ZC)])
        if tail:
            pltpu.sync_copy(zbuf_v.at[pl.ds(0, tail)],
                            acc_s.at[pl.ds(base + nfull * ZC, tail)])
        plsc.subcore_barrier()

        ebase = wid * EW

        def fire_idx(i, g):
            off = ebase + g * K
            pltpu.async_copy(src_hbm.at[pl.ds(off, K)], idxs_v[i], isem_s[i])
            pltpu.async_copy(dst_hbm.at[pl.ds(off, K)], idxd_v[i], isem_d[i])

        def wait_idx(i):
            dummy = src_hbm.at[pl.ds(ebase, K)]
            pltpu.make_async_copy(dummy, idxs_v[i], isem_s[i]).wait()
            pltpu.make_async_copy(dummy, idxd_v[i], isem_d[i]).wait()

        def fire_gather(b, i):
            pltpu.async_copy(tbl_hbm.at[idxs_v[i]], rows_v[b], sems[b])

        for i in range(NI):
            fire_idx(i, i)
        for b in range(NB):
            wait_idx(b)
            fire_gather(b, b)

        def outer(o, _):
            g0 = o * NI
            for j in range(NI):
                g = g0 + j
                b = j % NB

                @pl.when(g < G)
                def _():
                    pltpu.make_async_copy(tbl_hbm.at[idxs_v[j]], rows_v[b],
                                          sems[b]).wait()
                    pltpu.sync_copy(rows_v[b], acc_s.at[idxd_v[j]], add=True)

                    @pl.when(g + NB < G)
                    def _():
                        i2 = (j + NB) % NI
                        wait_idx(i2)
                        fire_gather(b, i2)

                    @pl.when(g + NI < G)
                    def _():
                        fire_idx(j, g + NI)
            return 0

        lax.fori_loop(0, (G + NI - 1) // NI, outer, 0)
        plsc.subcore_barrier()
        pltpu.sync_copy(acc_s.at[pl.ds(base, RPT)],
                        out_hbm.at[cid, pl.ds(base, RPT)])

    fn = pl.kernel(
        body,
        out_type=jax.ShapeDtypeStruct((NC, ndst_pad, D), f32),
        mesh=_sc_mesh(),
        scratch_types=(
            [pltpu.VMEM((K,), jnp.int32)] * (2 * NI)
            + [pltpu.VMEM((K, D), f32)] * NB
            + [pltpu.VMEM((ZC, D), f32),
               pltpu.VMEM_SHARED((ndst_pad, D), f32)]
            + [pltpu.SemaphoreType.DMA] * (NB + 2 * NI)
        ),
    )
    return fn(table, src, dst)


def _sc_pass_dual(tblA, eflat, E, npadA, tblB, srcB, dstB, npadB):
    D = tblA.shape[1]
    EW = E // NS
    K = 80
    NB = 4
    NI = 2 * NB
    G = EW // K
    assert EW % K == 0
    ZC = 40
    f32 = jnp.float32

    def body(tA, sA, tB, sB2, dB, outA, outB, *scr):
        idxs_v = scr[0:NI]
        idxd_v = scr[NI:2 * NI]
        rows_v = scr[2 * NI:2 * NI + NB]
        zbuf_v = scr[2 * NI + NB]
        acc_s = scr[2 * NI + NB + 1]
        p = 2 * NI + NB + 2
        sems = scr[p:p + NB]
        isem_s = scr[p + NB:p + NB + NI]
        isem_d = scr[p + NB + NI:p + NB + 2 * NI]
        cid = lax.axis_index("c")
        sid = lax.axis_index("s")
        zero16 = jnp.zeros((L,), f32)

        def zf(i, _):
            r = i // (D // L)
            c = i % (D // L)
            zbuf_v[r, pl.ds(c * L, L)] = zero16
            return 0

        lax.fori_loop(0, ZC * (D // L), zf, 0)

        def run(tbl_hbm, src_hbm, soff, dst_hbm, doff, out_hbm, RPT):
            base = sid * RPT
            for q in range(RPT // ZC):
                pltpu.sync_copy(zbuf_v, acc_s.at[pl.ds(base + q * ZC, ZC)])
            plsc.subcore_barrier()

            ebase = sid * EW

            def fire_idx(i, g):
                off = ebase + g * K
                pltpu.async_copy(src_hbm.at[pl.ds(soff + off, K)], idxs_v[i],
                                 isem_s[i])
                pltpu.async_copy(dst_hbm.at[pl.ds(doff + off, K)], idxd_v[i],
                                 isem_d[i])

            def wait_idx(i):
                dummy = src_hbm.at[pl.ds(soff + ebase, K)]
                pltpu.make_async_copy(dummy, idxs_v[i], isem_s[i]).wait()
                pltpu.make_async_copy(dummy, idxd_v[i], isem_d[i]).wait()

            def fire_gather(b, i):
                pltpu.async_copy(tbl_hbm.at[idxs_v[i]], rows_v[b], sems[b])

            for i in range(NI):
                fire_idx(i, i)
            for b in range(NB):
                wait_idx(b)
                fire_gather(b, b)

            def outer(o, _):
                g0 = o * NI
                for j in range(NI):
                    g = g0 + j
                    b = j % NB

                    @pl.when(g < G)
                    def _():
                        pltpu.make_async_copy(tbl_hbm.at[idxs_v[j]],
                                              rows_v[b], sems[b]).wait()
                        pltpu.sync_copy(rows_v[b], acc_s.at[idxd_v[j]],
                                        add=True)

                        @pl.when(g + NB < G)
                        def _():
                            i2 = (j + NB) % NI
                            wait_idx(i2)
                            fire_gather(b, i2)

                        @pl.when(g + NI < G)
                        def _():
                            fire_idx(j, g + NI)
                return 0

            lax.fori_loop(0, (G + NI - 1) // NI, outer, 0)
            plsc.subcore_barrier()
            pltpu.sync_copy(acc_s.at[pl.ds(base, RPT)],
                            out_hbm.at[pl.ds(base, RPT)])

        @pl.when(cid == 0)
        def _():
            run(tA, sA, 0, sA, E, outA, npadA // NS)

        @pl.when(cid == 1)
        def _():
            run(tB, sB2, 0, dB, 0, outB, npadB // NS)

    fn = pl.kernel(
        body,
        out_type=(jax.ShapeDtypeStruct((npadA, D), f32),
                  jax.ShapeDtypeStruct((npadB, D), f32)),
        mesh=_sc_mesh(),
        scratch_types=(
            [pltpu.VMEM((K,), jnp.int32)] * (2 * NI)
            + [pltpu.VMEM((K, D), f32)] * NB
            + [pltpu.VMEM((ZC, D), f32),
               pltpu.VMEM_SHARED((npadA, D), f32)]
            + [pltpu.SemaphoreType.DMA] * (NB + 2 * NI)
        ),
    )
    return fn(tblA, eflat, tblB, srcB, dstB)


def _tc_linear(X, Wt, b2):
    N, Din = X.shape
    Dout = Wt.shape[1]
    BN = 1000

    def body(x_ref, w_ref, b_ref, o_ref):
        o_ref[...] = (jnp.dot(x_ref[...], w_ref[...],
                              preferred_element_type=jnp.float32)
                      + b_ref[...])

    return pl.pallas_call(
        body,
        grid=(N // BN,),
        in_specs=[pl.BlockSpec((BN, Din), lambda i: (i, 0)),
                  pl.BlockSpec((Din, Dout), lambda i: (0, 0)),
                  pl.BlockSpec((1, Dout), lambda i: (0, 0))],
        out_specs=pl.BlockSpec((BN, Dout), lambda i: (i, 0)),
        out_shape=jax.ShapeDtypeStruct((N, Dout), jnp.float32),
    )(X, Wt, b2)


def _tc_prescale(Xl, degp2, dvp2):
    N, D = Xl.shape
    BN = 1000

    def body(x_ref, d0, d1, v0, v1, o1, o2, ac, dic, dvc):
        deg = d0[0] + d1[0] + 1.0
        a = lax.rsqrt(deg)
        di = 1.0 / deg
        dv = v0[0] + v1[0]
        dvi = jnp.where(dv > 0, lax.rsqrt(jnp.maximum(dv, 1.0)), 0.0)
        x = x_ref[...]
        o1[...] = x * a
        o2[...] = x * dvi
        ac[...] = a
        dic[...] = di
        dvc[...] = dvi

    row = pl.BlockSpec((BN, D), lambda i: (i, 0))
    col = pl.BlockSpec((BN, 1), lambda i: (i, 0))
    colp = pl.BlockSpec((1, BN, 1), lambda i: (0, i, 0))
    colp2 = pl.BlockSpec((1, BN, 1), lambda i: (1, i, 0))
    rd = jax.ShapeDtypeStruct((N, D), jnp.float32)
    cd = jax.ShapeDtypeStruct((N, 1), jnp.float32)
    return pl.pallas_call(
        body,
        grid=(N // BN,),
        in_specs=[row, colp, colp2, colp, colp2],
        out_specs=[row, row, col, col, col],
        out_shape=[rd, rd, cd, cd, cd],
    )(Xl, degp2, degp2, dvp2, dvp2)


def _tc_ze(z, dep2):
    Mp, D = z.shape
    BM = 1024

    def body(z_ref, d0, d1, o_ref):
        de = d0[0] + d1[0]
        dei = jnp.where(de > 0, 1.0 / jnp.maximum(de, 1.0), 0.0)
        o_ref[...] = z_ref[...] * dei

    colp = pl.BlockSpec((1, BM, 1), lambda i: (0, i, 0))
    colp2 = pl.BlockSpec((1, BM, 1), lambda i: (1, i, 0))
    return pl.pallas_call(
        body,
        grid=(Mp // BM,),
        in_specs=[pl.BlockSpec((BM, D), lambda i: (i, 0)), colp, colp2],
        out_specs=pl.BlockSpec((BM, D), lambda i: (i, 0)),
        out_shape=jax.ShapeDtypeStruct((Mp, D), jnp.float32),
    )(z, dep2, dep2)


def _tc_final(g, hgp, Xl, a_col, di_col, dvi_col):
    N, D = Xl.shape
    BN = 1000

    def body(gr, h0r, h1r, xr, ar, dir_, dvr, o_ref):
        agg = gr[...] * ar[...]
        hg = (h0r[0] + h1r[0]) * dvr[...]
        self_term = xr[...] * dir_[...]
        o_ref[...] = jnp.maximum(0.5 * (agg + self_term + hg), 0.0)

    row = pl.BlockSpec((BN, D), lambda i: (i, 0))
    col = pl.BlockSpec((BN, 1), lambda i: (i, 0))
    return pl.pallas_call(
        body,
        grid=(N // BN,),
        in_specs=[row,
                  pl.BlockSpec((1, BN, D), lambda i: (0, i, 0)),
                  pl.BlockSpec((1, BN, D), lambda i: (1, i, 0)),
                  row, col, col, col],
        out_specs=row,
        out_shape=jax.ShapeDtypeStruct((N, D), jnp.float32),
    )(g, hgp, hgp, Xl, a_col, di_col, dvi_col)


def kernel(X, edge_index, hyper_node_idx, hyper_edge_idx, W, b):
    N, Din = X.shape
    Dout = W.shape[0]
    M = 5000
    E = edge_index.shape[1]
    n_pad = ((N + NS * L - 1) // (NS * L)) * (NS * L)
    m_pad = ((M + 1024 - 1) // 1024) * 1024

    eflat = edge_index.reshape(2 * E)

    Xl = _tc_linear(X, W.T, b[None, :])
    degp, dvp, dep = _sc_degrees(eflat, hyper_node_idx, hyper_edge_idx,
                                 E, n_pad, m_pad)

    Xla, Y, a_col, di_col, dvi_col = _tc_prescale(
        Xl, degp.reshape(NC, n_pad, 1), dvp.reshape(NC, n_pad, 1))

    agg, zraw = _sc_pass_dual(Xla, eflat, E, n_pad,
                              Y, hyper_node_idx, hyper_edge_idx, m_pad)
    Ze = _tc_ze(zraw, dep.reshape(NC, m_pad, 1))
    hgp = _sc_pass(Ze, hyper_edge_idx, hyper_node_idx, n_pad)

    return _tc_final(agg, hgp, Xl, a_col, di_col, dvi_col)


# speedup vs baseline: 1.0994x; 1.0994x over previous
import jax
import jax.numpy as jnp
from jax import lax
from jax.experimental import pallas as pl
from jax.experimental.pallas import tpu as pltpu
from jax.experimental.pallas import tpu_sc as plsc

NC = 2
NS = 16
NW = NC * NS
L = 16


def _sc_mesh():
    return plsc.VectorSubcoreMesh(core_axis_name="c", subcore_axis_name="s")


def _sc_degrees(eflat, hni, hei, E, n_pad, m_pad):
    EW = E // NW
    K = 80
    NB = 5
    assert EW % K == 0 and (EW // K) % NB == 0
    CN = n_pad // NS
    CM = m_pad // NS
    assert CN % 8 == 0 and CM % 8 == 0
    f32 = jnp.float32

    def body(eflat_hbm, hni_hbm, hei_hbm, deg_out, dv_out, de_out, *scr):
        idx_v = scr[0:NB]
        isem = scr[NB:2 * NB]
        ones_v, zero_v, s_n, s_v, s_e = scr[2 * NB:]
        cid = lax.axis_index("c")
        sid = lax.axis_index("s")
        wid = sid * NC + cid
        zero16 = jnp.zeros((L,), f32)
        one16 = jnp.ones((L,), f32)

        def fill(ref, n, vec):
            def f(i, _):
                ref[pl.ds(i * L, L)] = vec
                return 0
            lax.fori_loop(0, n // L, f, 0)

        fill(ones_v, K, one16)
        fill(zero_v, CN, zero16)

        pltpu.sync_copy(zero_v.at[pl.ds(0, CN)], s_n.at[pl.ds(sid * CN, CN)])
        pltpu.sync_copy(zero_v.at[pl.ds(0, CN)], s_v.at[pl.ds(sid * CN, CN)])
        pltpu.sync_copy(zero_v.at[pl.ds(0, CM)], s_e.at[pl.ds(sid * CM, CM)])
        plsc.subcore_barrier()

        base = wid * EW
        G = EW // K

        def count(idx_hbm, off0, shared):
            def fetch(b, g):
                pltpu.async_copy(idx_hbm.at[pl.ds(off0 + base + g * K, K)],
                                 idx_v[b], isem[b])

            for b in range(NB):
                fetch(b, b)

            def outer(o, _):
                for b in range(NB):
                    g = o * NB + b
                    pltpu.make_async_copy(idx_hbm.at[pl.ds(off0 + base, K)],
                                          idx_v[b], isem[b]).wait()
                    pltpu.sync_copy(ones_v, shared.at[idx_v[b]], add=True)

                    @pl.when(g + NB < G)
                    def _():
                        fetch(b, g + NB)
                return 0

            lax.fori_loop(0, G // NB, outer, 0)

        count(eflat_hbm, E, s_n)
        count(hni_hbm, 0, s_v)
        count(hei_hbm, 0, s_e)
        plsc.subcore_barrier()

        def copy_out(shared, out_ref, cw, pitch):
            pltpu.sync_copy(shared.at[pl.ds(sid * cw, cw)],
                            zero_v.at[pl.ds(0, cw)])
            pltpu.sync_copy(zero_v.at[pl.ds(0, cw)],
                            out_ref.at[pl.ds(cid * pitch + sid * cw, cw)])

        copy_out(s_n, deg_out, CN, n_pad)
        copy_out(s_v, dv_out, CN, n_pad)
        copy_out(s_e, de_out, CM, m_pad)

    fn = pl.kernel(
        body,
        out_type=(jax.ShapeDtypeStruct((NC * n_pad,), f32),
                  jax.ShapeDtypeStruct((NC * n_pad,), f32),
                  jax.ShapeDtypeStruct((NC * m_pad,), f32)),
        mesh=_sc_mesh(),
        scratch_types=(
            [pltpu.VMEM((K,), jnp.int32)] * NB
            + [pltpu.SemaphoreType.DMA] * NB
            + [pltpu.VMEM((K,), f32),
               pltpu.VMEM((CN,), f32),
               pltpu.VMEM_SHARED((n_pad,), f32),
               pltpu.VMEM_SHARED((n_pad,), f32),
               pltpu.VMEM_SHARED((m_pad,), f32)]
        ),
    )
    return fn(eflat, hni, hei)


def _sc_pass(table, src, dst, ndst_pad):
    nsrc, D = table.shape
    E = src.shape[0]
    EW = E // NW
    K = 80
    NB = 4
    G = EW // K
    GOUT = (G + NB - 1) // NB
    assert EW % K == 0
    RPT = ndst_pad // NS
    ZC = min(RPT, 40)
    nfull, tail = RPT // ZC, RPT % ZC
    f32 = jnp.float32

    NI = 2 * NB

    def body(tbl_hbm, src_hbm, dst_hbm, out_hbm, *scr):
        idxs_v = scr[0:NI]
        idxd_v = scr[NI:2 * NI]
        rows_v = scr[2 * NI:2 * NI + NB]
        zbuf_v = scr[2 * NI + NB]
        acc_s = scr[2 * NI + NB + 1]
        p = 2 * NI + NB + 2
        sems = scr[p:p + NB]
        isem_s = scr[p + NB:p + NB + NI]
        isem_d = scr[p + NB + NI:p + NB + 2 * NI]
        cid = lax.axis_index("c")
        sid = lax.axis_index("s")
        wid = sid * NC + cid
        zero16 = jnp.zeros((L,), f32)

        def zf(i, _):
            r = i // (D // L)
            c = i % (D // L)
            zbuf_v[r, pl.ds(c * L, L)] = zero16
            return 0

        lax.fori_loop(0, ZC * (D // L), zf, 0)

        base = sid * RPT
        for q in range(nfull):
            pltpu.sync_copy(zbuf_v, acc_s.at[pl.ds(base + q * ZC, ZC)])
        if tail:
            pltpu.sync_copy(zbuf_v.at[pl.ds(0, tail)],
                            acc_s.at[pl.ds(base + nfull * ZC, tail)])
        plsc.subcore_barrier()

        ebase = wid * EW

        def fire_idx(i, g):
            off = ebase + g * K
            pltpu.async_copy(src_hbm.at[pl.ds(off, K)], idxs_v[i], isem_s[i])
            pltpu.async_copy(dst_hbm.at[pl.ds(off, K)], idxd_v[i], isem_d[i])

        def wait_idx(i):
            dummy = src_hbm.at[pl.ds(ebase, K)]
            pltpu.make_async_copy(dummy, idxs_v[i], isem_s[i]).wait()
            pltpu.make_async_copy(dummy, idxd_v[i], isem_d[i]).wait()

        def fire_gather(b, i):
            pltpu.async_copy(tbl_hbm.at[idxs_v[i]], rows_v[b], sems[b])

        for i in range(NI):
            fire_idx(i, i)
        for b in range(NB):
            wait_idx(b)
            fire_gather(b, b)

        def outer(o, _):
            g0 = o * NI
            for j in range(NI):
                g = g0 + j
                b = j % NB

                @pl.when(g < G)
                def _():
                    pltpu.make_async_copy(tbl_hbm.at[idxs_v[j]], rows_v[b],
                                          sems[b]).wait()
                    pltpu.sync_copy(rows_v[b], acc_s.at[idxd_v[j]], add=True)

                    @pl.when(g + NB < G)
                    def _():
                        i2 = (j + NB) % NI
                        wait_idx(i2)
                        fire_gather(b, i2)

                    @pl.when(g + NI < G)
                    def _():
                        fire_idx(j, g + NI)
            return 0

        lax.fori_loop(0, (G + NI - 1) // NI, outer, 0)
        plsc.subcore_barrier()
        pltpu.sync_copy(acc_s.at[pl.ds(base, RPT)],
                        out_hbm.at[cid, pl.ds(base, RPT)])

    fn = pl.kernel(
        body,
        out_type=jax.ShapeDtypeStruct((NC, ndst_pad, D), f32),
        mesh=_sc_mesh(),
        scratch_types=(
            [pltpu.VMEM((K,), jnp.int32)] * (2 * NI)
            + [pltpu.VMEM((K, D), f32)] * NB
            + [pltpu.VMEM((ZC, D), f32),
               pltpu.VMEM_SHARED((ndst_pad, D), f32)]
            + [pltpu.SemaphoreType.DMA] * (NB + 2 * NI)
        ),
    )
    return fn(table, src, dst)


def _sc_pass_dual(tblA, eflat, E, npadA, tblB, srcB, dstB, npadB):
    D = tblA.shape[1]
    EW = E // NS
    K = 80
    NB = 4
    NI = 2 * NB
    G = EW // K
    assert EW % K == 0
    ZC = 40
    f32 = jnp.float32

    def body(tA, sA, tB, sB2, dB, outA, outB, *scr):
        idxs_v = scr[0:NI]
        idxd_v = scr[NI:2 * NI]
        rows_v = scr[2 * NI:2 * NI + NB]
        zbuf_v = scr[2 * NI + NB]
        acc_s = scr[2 * NI + NB + 1]
        p = 2 * NI + NB + 2
        sems = scr[p:p + NB]
        isem_s = scr[p + NB:p + NB + NI]
        isem_d = scr[p + NB + NI:p + NB + 2 * NI]
        cid = lax.axis_index("c")
        sid = lax.axis_index("s")
        zero16 = jnp.zeros((L,), f32)

        def zf(i, _):
            r = i // (D // L)
            c = i % (D // L)
            zbuf_v[r, pl.ds(c * L, L)] = zero16
            return 0

        lax.fori_loop(0, ZC * (D // L), zf, 0)

        def run(tbl_hbm, src_hbm, soff, dst_hbm, doff, out_hbm, RPT):
            base = sid * RPT
            for q in range(RPT // ZC):
                pltpu.sync_copy(zbuf_v, acc_s.at[pl.ds(base + q * ZC, ZC)])
            plsc.subcore_barrier()

            ebase = sid * EW

            def fire_idx(i, g):
                off = ebase + g * K
                pltpu.async_copy(src_hbm.at[pl.ds(soff + off, K)], idxs_v[i],
                                 isem_s[i])
                pltpu.async_copy(dst_hbm.at[pl.ds(doff + off, K)], idxd_v[i],
                                 isem_d[i])

            def wait_idx(i):
                dummy = src_hbm.at[pl.ds(soff + ebase, K)]
                pltpu.make_async_copy(dummy, idxs_v[i], isem_s[i]).wait()
                pltpu.make_async_copy(dummy, idxd_v[i], isem_d[i]).wait()

            def fire_gather(b, i):
                pltpu.async_copy(tbl_hbm.at[idxs_v[i]], rows_v[b], sems[b])

            for i in range(NI):
                fire_idx(i, i)
            for b in range(NB):
                wait_idx(b)
                fire_gather(b, b)

            def outer(o, _):
                g0 = o * NI
                for j in range(NI):
                    g = g0 + j
                    b = j % NB

                    @pl.when(g < G)
                    def _():
                        pltpu.make_async_copy(tbl_hbm.at[idxs_v[j]],
                                              rows_v[b], sems[b]).wait()
                        pltpu.sync_copy(rows_v[b], acc_s.at[idxd_v[j]],
                                        add=True)

                        @pl.when(g + NB < G)
                        def _():
                            i2 = (j + NB) % NI
                            wait_idx(i2)
                            fire_gather(b, i2)

                        @pl.when(g + NI < G)
                        def _():
                            fire_idx(j, g + NI)
                return 0

            lax.fori_loop(0, (G + NI - 1) // NI, outer, 0)
            plsc.subcore_barrier()
            pltpu.sync_copy(acc_s.at[pl.ds(base, RPT)],
                            out_hbm.at[pl.ds(base, RPT)])

        @pl.when(cid == 0)
        def _():
            run(tA, sA, 0, sA, E, outA, npadA // NS)

        @pl.when(cid == 1)
        def _():
            run(tB, sB2, 0, dB, 0, outB, npadB // NS)

    fn = pl.kernel(
        body,
        out_type=(jax.ShapeDtypeStruct((npadA, D), f32),
                  jax.ShapeDtypeStruct((npadB, D), f32)),
        mesh=_sc_mesh(),
        scratch_types=(
            [pltpu.VMEM((K,), jnp.int32)] * (2 * NI)
            + [pltpu.VMEM((K, D), f32)] * NB
            + [pltpu.VMEM((ZC, D), f32),
               pltpu.VMEM_SHARED((npadA, D), f32)]
            + [pltpu.SemaphoreType.DMA] * (NB + 2 * NI)
        ),
    )
    return fn(tblA, eflat, tblB, srcB, dstB)


def _tc_linear(X, Wt, b2):
    N, Din = X.shape
    Dout = Wt.shape[1]
    BN = 1000

    def body(x_ref, w_ref, b_ref, o_ref):
        o_ref[...] = (jnp.dot(x_ref[...], w_ref[...],
                              preferred_element_type=jnp.float32)
                      + b_ref[...])

    return pl.pallas_call(
        body,
        grid=(N // BN,),
        in_specs=[pl.BlockSpec((BN, Din), lambda i: (i, 0)),
                  pl.BlockSpec((Din, Dout), lambda i: (0, 0)),
                  pl.BlockSpec((1, Dout), lambda i: (0, 0))],
        out_specs=pl.BlockSpec((BN, Dout), lambda i: (i, 0)),
        out_shape=jax.ShapeDtypeStruct((N, Dout), jnp.float32),
    )(X, Wt, b2)


def _tc_prescale(Xl, degp, dvp, n_pad):
    N, D = Xl.shape
    BN = 1024
    NG = n_pad // BN

    def body(x_ref, d0, d1, v0, v1, o1, o2, ac, dic, dvc):
        deg = d0[...] + d1[...] + 1.0
        a = lax.rsqrt(deg)
        di = 1.0 / deg
        dv = v0[...] + v1[...]
        dvi = jnp.where(dv > 0, lax.rsqrt(jnp.maximum(dv, 1.0)), 0.0)
        x = x_ref[...]
        o1[...] = x * a[:, None]
        o2[...] = x * dvi[:, None]
        ac[...] = a
        dic[...] = di
        dvc[...] = dvi

    row = pl.BlockSpec((BN, D), lambda i: (i, 0))
    c0 = pl.BlockSpec((BN,), lambda i: (i,))
    c1 = pl.BlockSpec((BN,), lambda i: (NG + i,))
    rd = jax.ShapeDtypeStruct((N, D), jnp.float32)
    cd = jax.ShapeDtypeStruct((N,), jnp.float32)
    return pl.pallas_call(
        body,
        grid=(NG,),
        in_specs=[row, c0, c1, c0, c1],
        out_specs=[row, row, c0, c0, c0],
        out_shape=[rd, rd, cd, cd, cd],
    )(Xl, degp, degp, dvp, dvp)


def _tc_ze(z, dep, m_pad):
    Mp, D = z.shape
    BM = 1024
    MG = m_pad // BM

    def body(z_ref, d0, d1, o_ref):
        de = d0[...] + d1[...]
        dei = jnp.where(de > 0, 1.0 / jnp.maximum(de, 1.0), 0.0)
        o_ref[...] = z_ref[...] * dei[:, None]

    c0 = pl.BlockSpec((BM,), lambda i: (i,))
    c1 = pl.BlockSpec((BM,), lambda i: (MG + i,))
    return pl.pallas_call(
        body,
        grid=(MG,),
        in_specs=[pl.BlockSpec((BM, D), lambda i: (i, 0)), c0, c1],
        out_specs=pl.BlockSpec((BM, D), lambda i: (i, 0)),
        out_shape=jax.ShapeDtypeStruct((Mp, D), jnp.float32),
    )(z, dep, dep)


def _tc_final(g, hgp, Xl, a_col, di_col, dvi_col):
    N, D = Xl.shape
    BN = 1024
    NG = (N + BN - 1) // BN

    def body(gr, h0r, h1r, xr, ar, dir_, dvr, o_ref):
        agg = gr[...] * ar[...][:, None]
        hg = (h0r[0] + h1r[0]) * dvr[...][:, None]
        self_term = xr[...] * dir_[...][:, None]
        o_ref[...] = jnp.maximum(0.5 * (agg + self_term + hg), 0.0)

    row = pl.BlockSpec((BN, D), lambda i: (i, 0))
    col = pl.BlockSpec((BN,), lambda i: (i,))
    return pl.pallas_call(
        body,
        grid=(NG,),
        in_specs=[row,
                  pl.BlockSpec((1, BN, D), lambda i: (0, i, 0)),
                  pl.BlockSpec((1, BN, D), lambda i: (1, i, 0)),
                  row, col, col, col],
        out_specs=row,
        out_shape=jax.ShapeDtypeStruct((N, D), jnp.float32),
    )(g, hgp, hgp, Xl, a_col, di_col, dvi_col)


def kernel(X, edge_index, hyper_node_idx, hyper_edge_idx, W, b):
    N, Din = X.shape
    Dout = W.shape[0]
    M = 5000
    E = edge_index.shape[1]
    n_pad = ((N + NS * L - 1) // (NS * L)) * (NS * L)
    m_pad = ((M + 1024 - 1) // 1024) * 1024

    eflat = edge_index.reshape(2 * E)

    Xl = _tc_linear(X, W.T, b[None, :])
    degp, dvp, dep = _sc_degrees(eflat, hyper_node_idx, hyper_edge_idx,
                                 E, n_pad, m_pad)

    Xla, Y, a_col, di_col, dvi_col = _tc_prescale(Xl, degp, dvp, n_pad)

    agg, zraw = _sc_pass_dual(Xla, eflat, E, n_pad,
                              Y, hyper_node_idx, hyper_edge_idx, m_pad)
    Ze = _tc_ze(zraw, dep, m_pad)
    hgp = _sc_pass(Ze, hyper_edge_idx, hyper_node_idx, n_pad)

    return _tc_final(agg, hgp, Xl, a_col, di_col, dvi_col)
